# baseline (speedup 1.0000x reference)
import jax
import jax.numpy as jnp
from jax import lax
from jax.experimental import pallas as pl
from jax.experimental.pallas import tpu as pltpu


def _fused_kernel(xt_ref, wbig_ref, mbias_ref, wcomb_ref, bcomb_ref,
                  xo_ref, po_ref, wbig_s, wcomb_s, mb_s, bc_s,
                  wbig_stg, wcomb_stg, mb_stg, bc_stg, dma_sem):
    Tb, SWD, B = xt_ref.shape
    SD = wbig_ref.shape[1]
    OL = wcomb_ref.shape[1]
    WD = 128
    n_groups = Tb * SWD // WD

    @pl.when(pl.program_id(0) == 0)
    def _prep():
        cps = [pltpu.make_async_copy(wbig_ref, wbig_stg, dma_sem.at[0]),
               pltpu.make_async_copy(wcomb_ref, wcomb_stg, dma_sem.at[1]),
               pltpu.make_async_copy(mbias_ref, mb_stg, dma_sem.at[2]),
               pltpu.make_async_copy(bcomb_ref, bc_stg, dma_sem.at[3])]
        for cp in cps:
            cp.start()
        for cp in cps:
            cp.wait()
        wbig_s[...] = wbig_stg[...].T.astype(jnp.bfloat16)
        wcomb_s[...] = wcomb_stg[...].T.astype(jnp.bfloat16)
        mb_s[...] = mb_stg[...].T
        bc_s[...] = bc_stg[...].T

    xt = xt_ref[...]

    x3 = xt.reshape(n_groups, WD, B)
    s1 = jnp.sum(x3, axis=1, keepdims=True)
    s2 = jnp.sum(x3 * x3, axis=1, keepdims=True)
    mean = s1 * (1.0 / WD)
    var = s2 * (1.0 / WD) - mean * mean
    rstd = lax.rsqrt(var + 1e-5)
    scale = rstd.reshape(Tb, SWD // WD, 1, B)
    shift = (mean * rstd).reshape(Tb, SWD // WD, 1, B)
    x4 = x3.reshape(Tb, SWD // WD, WD, B)
    nb2 = jnp.concatenate(
        [(x4[t] * scale[t] - shift[t]).reshape(SWD, B) for t in range(Tb)],
        axis=1).astype(jnp.bfloat16)

    xm2 = (jnp.dot(wbig_s[...], nb2, preferred_element_type=jnp.float32)
           + mb_s[...])
    po2 = (jnp.dot(wcomb_s[...], xm2.astype(jnp.bfloat16),
                   preferred_element_type=jnp.float32)
           + bc_s[...])

    for t in range(Tb):
        xo_ref[t, :, :] = xm2[:, t * B:(t + 1) * B]
        po_ref[t, :, :] = po2[:, t * B:(t + 1) * B]


def kernel(x, b_rep_node, Pm, pool_avg, pool_t, wbig_g, mbias, w_comb, b_comb):
    del b_rep_node, Pm, pool_avg, pool_t
    B, T, seg, D = x.shape
    SWD = seg * D
    SD_pad = wbig_g.shape[1]
    OL_pad = w_comb.shape[1]
    S = SD_pad // D
    O = 16
    L = OL_pad // O

    xt = jnp.transpose(x, (1, 2, 3, 0)).reshape(T, SWD, B)

    Tb = 8
    n_blocks = T // Tb

    xo_t, po_t = pl.pallas_call(
        _fused_kernel,
        out_shape=(jax.ShapeDtypeStruct((T, SD_pad, B), jnp.float32),
                   jax.ShapeDtypeStruct((T, OL_pad, B), jnp.float32)),
        grid_spec=pltpu.PrefetchScalarGridSpec(
            num_scalar_prefetch=0,
            grid=(n_blocks,),
            in_specs=[
                pl.BlockSpec((Tb, SWD, B), lambda r: (r, 0, 0)),
                pl.BlockSpec(memory_space=pl.ANY),
                pl.BlockSpec(memory_space=pl.ANY),
                pl.BlockSpec(memory_space=pl.ANY),
                pl.BlockSpec(memory_space=pl.ANY),
            ],
            out_specs=(pl.BlockSpec((Tb, SD_pad, B), lambda r: (r, 0, 0)),
                       pl.BlockSpec((Tb, OL_pad, B), lambda r: (r, 0, 0))),
            scratch_shapes=[
                pltpu.VMEM((SD_pad, SWD), jnp.bfloat16),
                pltpu.VMEM((OL_pad, SD_pad), jnp.bfloat16),
                pltpu.VMEM((SD_pad, 1), jnp.float32),
                pltpu.VMEM((OL_pad, 1), jnp.float32),
                pltpu.VMEM((SWD, SD_pad), jnp.float32),
                pltpu.VMEM((SD_pad, OL_pad), jnp.float32),
                pltpu.VMEM((1, SD_pad), jnp.float32),
                pltpu.VMEM((1, OL_pad), jnp.float32),
                pltpu.SemaphoreType.DMA((4,)),
            ],
        ),
        compiler_params=pltpu.CompilerParams(
            dimension_semantics=("arbitrary",),
            vmem_limit_bytes=60 * 1024 * 1024),
    )(xt, wbig_g, mbias, w_comb, b_comb)

    x_out = jnp.transpose(xo_t.reshape(T, S, D, B), (3, 0, 1, 2))
    layer_predict = jnp.transpose(po_t.reshape(T, O, L, B), (3, 0, 1, 2))
    return x_out, layer_predict


# speedup vs baseline: 1.0461x; 1.0461x over previous
import jax
import jax.numpy as jnp
from jax import lax
from jax.experimental import pallas as pl
from jax.experimental.pallas import tpu as pltpu


def _fused_kernel(xt_ref, wbig_ref, mbias_ref, wcomb_ref, bcomb_ref,
                  xo_ref, po_ref, wbig_s, wcomb_s, mb_s, bc_s):
    Tb, SWD, B = xt_ref.shape
    SD = wbig_ref.shape[1]
    OL = wcomb_ref.shape[1]
    WD = 128
    n_groups = Tb * SWD // WD

    @pl.when(pl.program_id(0) == 0)
    def _prep():
        wbig_s[...] = wbig_ref[...].T.astype(jnp.bfloat16)
        wcomb_s[...] = wcomb_ref[...].T.astype(jnp.bfloat16)
        mb_s[...] = mbias_ref[...].T
        bc_s[...] = bcomb_ref[...].T

    xt = xt_ref[...]

    x3 = xt.reshape(n_groups, WD, B)
    s1 = jnp.sum(x3, axis=1, keepdims=True)
    s2 = jnp.sum(x3 * x3, axis=1, keepdims=True)
    mean = s1 * (1.0 / WD)
    var = s2 * (1.0 / WD) - mean * mean
    rstd = lax.rsqrt(var + 1e-5)
    scale = rstd.reshape(Tb, SWD // WD, 1, B)
    shift = (mean * rstd).reshape(Tb, SWD // WD, 1, B)
    x4 = x3.reshape(Tb, SWD // WD, WD, B)
    nb2 = jnp.concatenate(
        [(x4[t] * scale[t] - shift[t]).reshape(SWD, B) for t in range(Tb)],
        axis=1).astype(jnp.bfloat16)

    xm2 = (jnp.dot(wbig_s[...], nb2, preferred_element_type=jnp.float32)
           + mb_s[...])
    po2 = (jnp.dot(wcomb_s[...], xm2.astype(jnp.bfloat16),
                   preferred_element_type=jnp.float32)
           + bc_s[...])

    for t in range(Tb):
        xo_ref[t, :, :] = xm2[:, t * B:(t + 1) * B]
        po_ref[t, :, :] = po2[:, t * B:(t + 1) * B]


def kernel(x, b_rep_node, Pm, pool_avg, pool_t, wbig_g, mbias, w_comb, b_comb):
    del b_rep_node, Pm, pool_avg, pool_t
    B, T, seg, D = x.shape
    SWD = seg * D
    SD_pad = wbig_g.shape[1]
    OL_pad = w_comb.shape[1]
    S = SD_pad // D
    O = 16
    L = OL_pad // O

    xt = jnp.transpose(x, (1, 2, 3, 0)).reshape(T, SWD, B)

    Tb = 8
    n_blocks = T // Tb

    xo_t, po_t = pl.pallas_call(
        _fused_kernel,
        out_shape=(jax.ShapeDtypeStruct((T, SD_pad, B), jnp.float32),
                   jax.ShapeDtypeStruct((T, OL_pad, B), jnp.float32)),
        grid_spec=pltpu.PrefetchScalarGridSpec(
            num_scalar_prefetch=0,
            grid=(n_blocks,),
            in_specs=[
                pl.BlockSpec((Tb, SWD, B), lambda r: (r, 0, 0)),
                pl.BlockSpec((SWD, SD_pad), lambda r: (0, 0)),
                pl.BlockSpec((1, SD_pad), lambda r: (0, 0)),
                pl.BlockSpec((SD_pad, OL_pad), lambda r: (0, 0)),
                pl.BlockSpec((1, OL_pad), lambda r: (0, 0)),
            ],
            out_specs=(pl.BlockSpec((Tb, SD_pad, B), lambda r: (r, 0, 0)),
                       pl.BlockSpec((Tb, OL_pad, B), lambda r: (r, 0, 0))),
            scratch_shapes=[
                pltpu.VMEM((SD_pad, SWD), jnp.bfloat16),
                pltpu.VMEM((OL_pad, SD_pad), jnp.bfloat16),
                pltpu.VMEM((SD_pad, 1), jnp.float32),
                pltpu.VMEM((OL_pad, 1), jnp.float32),
            ],
        ),
        compiler_params=pltpu.CompilerParams(
            dimension_semantics=("arbitrary",),
            vmem_limit_bytes=60 * 1024 * 1024),
    )(xt, wbig_g, mbias, w_comb, b_comb)

    x_out = jnp.transpose(xo_t.reshape(T, S, D, B), (3, 0, 1, 2))
    layer_predict = jnp.transpose(po_t.reshape(T, O, L, B), (3, 0, 1, 2))
    return x_out, layer_predict


# speedup vs baseline: 1.0894x; 1.0414x over previous
import jax
import jax.numpy as jnp
from jax import lax
from jax.experimental import pallas as pl
from jax.experimental.pallas import tpu as pltpu


def _fused_kernel(xt_ref, wbig_ref, mbias_ref, wcomb_ref, bcomb_ref,
                  xo_ref, po_ref, wbig_s, wcomb_s, mb_s, bc_s):
    Tb, SWD, B = xt_ref.shape
    SD = wbig_ref.shape[1]
    OL = wcomb_ref.shape[1]
    WD = 128
    n_groups = Tb * SWD // WD

    @pl.when(pl.program_id(0) == 0)
    def _prep():
        wbig_s[...] = wbig_ref[...].T.astype(jnp.bfloat16)
        wcomb_s[...] = wcomb_ref[...].T.astype(jnp.bfloat16)
        mb_s[...] = mbias_ref[...].T
        bc_s[...] = bcomb_ref[...].T

    xt = xt_ref[...]

    x3 = xt.reshape(n_groups, WD, B)
    s1 = jnp.sum(x3, axis=1, keepdims=True)
    s2 = jnp.sum(x3 * x3, axis=1, keepdims=True)
    mean = s1 * (1.0 / WD)
    var = s2 * (1.0 / WD) - mean * mean
    rstd = lax.rsqrt(var + 1e-5)
    scale = rstd.reshape(Tb, SWD // WD, 1, B)
    shift = (mean * rstd).reshape(Tb, SWD // WD, 1, B)
    x4 = x3.reshape(Tb, SWD // WD, WD, B)
    nb2 = jnp.concatenate(
        [(x4[t] * scale[t] - shift[t]).reshape(SWD, B) for t in range(Tb)],
        axis=1).astype(jnp.bfloat16)

    xm2 = (jnp.dot(wbig_s[...], nb2, preferred_element_type=jnp.float32)
           + mb_s[...])
    po2 = (jnp.dot(wcomb_s[...], xm2.astype(jnp.bfloat16),
                   preferred_element_type=jnp.float32)
           + bc_s[...])

    for t in range(Tb):
        xo_ref[t, :, :] = xm2[:, t * B:(t + 1) * B]
        po_ref[t, :, :] = po2[:, t * B:(t + 1) * B]


def kernel(x, b_rep_node, Pm, pool_avg, pool_t, wbig_g, mbias, w_comb, b_comb):
    del b_rep_node, Pm, pool_avg, pool_t
    B, T, seg, D = x.shape
    SWD = seg * D
    SD_pad = wbig_g.shape[1]
    OL_pad = w_comb.shape[1]
    S = SD_pad // D
    O = 16
    L = OL_pad // O

    xt = jnp.transpose(x, (1, 2, 3, 0)).reshape(T, SWD, B)

    Tb = 16
    n_blocks = T // Tb

    xo_t, po_t = pl.pallas_call(
        _fused_kernel,
        out_shape=(jax.ShapeDtypeStruct((T, SD_pad, B), jnp.float32),
                   jax.ShapeDtypeStruct((T, OL_pad, B), jnp.float32)),
        grid_spec=pltpu.PrefetchScalarGridSpec(
            num_scalar_prefetch=0,
            grid=(n_blocks,),
            in_specs=[
                pl.BlockSpec((Tb, SWD, B), lambda r: (r, 0, 0)),
                pl.BlockSpec((SWD, SD_pad), lambda r: (0, 0)),
                pl.BlockSpec((1, SD_pad), lambda r: (0, 0)),
                pl.BlockSpec((SD_pad, OL_pad), lambda r: (0, 0)),
                pl.BlockSpec((1, OL_pad), lambda r: (0, 0)),
            ],
            out_specs=(pl.BlockSpec((Tb, SD_pad, B), lambda r: (r, 0, 0)),
                       pl.BlockSpec((Tb, OL_pad, B), lambda r: (r, 0, 0))),
            scratch_shapes=[
                pltpu.VMEM((SD_pad, SWD), jnp.bfloat16),
                pltpu.VMEM((OL_pad, SD_pad), jnp.bfloat16),
                pltpu.VMEM((SD_pad, 1), jnp.float32),
                pltpu.VMEM((OL_pad, 1), jnp.float32),
            ],
        ),
        compiler_params=pltpu.CompilerParams(
            dimension_semantics=("arbitrary",),
            vmem_limit_bytes=60 * 1024 * 1024),
    )(xt, wbig_g, mbias, w_comb, b_comb)

    x_out = jnp.transpose(xo_t.reshape(T, S, D, B), (3, 0, 1, 2))
    layer_predict = jnp.transpose(po_t.reshape(T, O, L, B), (3, 0, 1, 2))
    return x_out, layer_predict
